# baseline (speedup 1.0000x reference)
import functools

import jax
import jax.numpy as jnp
from jax import lax
from jax.experimental import pallas as pl
from jax.experimental.pallas import tpu as pltpu
from jax.experimental.pallas import tpu_sc as plsc

BATCH = 4096
SEQ = 200
DIM = 64
NW = 32
BW = BATCH // NW
L = 16
NG = BW // L
NPB = 4
NOB = 2
AH = 3
OST = BW + 1
VOCAB_PAIRS = 500000


def _build():
    mesh = plsc.VectorSubcoreMesh(core_axis_name="c", subcore_axis_name="s")

    @functools.partial(
        pl.kernel,
        mesh=mesh,
        out_type=jax.ShapeDtypeStruct((SEQ, DIM, BATCH), jnp.float32),
        scratch_types=[
            pltpu.VMEM((SEQ, DIM), jnp.float32),
            pltpu.VMEM((BW,), jnp.int32),
            [pltpu.VMEM((BW,), jnp.int32) for _ in range(NPB)],
            [pltpu.VMEM((BW,), jnp.int32) for _ in range(NPB)],
            [pltpu.VMEM((BW, 2 * DIM), jnp.float32) for _ in range(NPB)],
            [pltpu.VMEM((DIM, OST), jnp.float32) for _ in range(NOB)],
            [pltpu.SemaphoreType.DMA for _ in range(NPB)],
            [pltpu.SemaphoreType.DMA for _ in range(NPB)],
            [pltpu.SemaphoreType.DMA for _ in range(NOB)],
        ],
        compiler_params=pltpu.CompilerParams(
            use_tc_tiling_on_sc=True, needs_layout_passes=False),
    )
    def emb(ids_hbm, table_hbm, pos_hbm, out_hbm, pos_v, par_v, raw_v,
            idx2_v, pair_v, obuf, isem, gsem, wsem):
        wid = lax.axis_index("s") * 2 + lax.axis_index("c")
        b0 = wid * BW
        pltpu.sync_copy(pos_hbm, pos_v)

        lanes = lax.iota(jnp.int32, L)
        ck = [lanes + L * k for k in range(DIM // L)]

        def fire_ids(s, slot):
            pltpu.async_copy(ids_hbm.at[s, pl.ds(b0, BW)], raw_v[slot],
                             isem[slot])

        def wait_ids(slot):
            pltpu.make_async_copy(ids_hbm.at[0, pl.ds(b0, BW)], raw_v[slot],
                                  isem[slot]).wait()

        def fire_gather(s, slot):
            for g in range(NG):
                t = raw_v[slot][pl.ds(L * g, L)]
                idx2_v[slot][pl.ds(L * g, L)] = lax.shift_right_logical(t, 1)
            pltpu.async_copy(table_hbm.at[idx2_v[slot]], pair_v[slot],
                             gsem[slot])

        def wait_gather(slot):
            pltpu.make_async_copy(
                table_hbm.at[idx2_v[slot]], pair_v[slot], gsem[slot]).wait()

        def wait_wb(slot):
            pltpu.make_async_copy(
                obuf[slot].at[:, pl.ds(0, BW)],
                out_hbm.at[0, :, pl.ds(b0, BW)], wsem[slot]
            ).wait()

        for s0 in range(NPB):
            fire_ids(s0, s0)
        for s0 in range(AH):
            wait_ids(s0)
            fire_gather(s0, s0)

        def iter_body(i, carry):
            for q in range(NPB):
                s = NPB * i + q
                wait_gather(q)

                for g in range(NG):
                    t = raw_v[q][pl.ds(L * g, L)]
                    par_v[pl.ds(L * g, L)] = jnp.bitwise_and(t, 1)

                gslot = (q + AH) % NPB

                @pl.when(s + AH < SEQ)
                def _():
                    wait_ids(gslot)
                    fire_gather(s + AH, gslot)

                @pl.when(s + NPB < SEQ)
                def _():
                    fire_ids(s + NPB, q)

                pk = [pos_v[s, pl.ds(L * k, L)] for k in range(DIM // L)]
                pv = pair_v[q]
                oq = q % NOB
                ob = obuf[oq]

                @pl.when(jnp.logical_or(i > 0, q >= NOB))
                def _():
                    wait_wb(oq)

                @plsc.parallel_loop(0, BW, step=1, unroll=8)
                def b_body(b):
                    bsplat = jnp.full((L,), b, jnp.int32)
                    pars = plsc.load_gather(par_v, (bsplat,))
                    c0 = lax.shift_left(pars, 6)
                    for k in range(DIM // L):
                        v = plsc.load_gather(pv, (bsplat, c0 + ck[k]))
                        plsc.store_scatter(ob, (ck[k], bsplat), v + pk[k])

                pltpu.async_copy(ob.at[:, pl.ds(0, BW)],
                                 out_hbm.at[s, :, pl.ds(b0, BW)], wsem[oq])
            return carry

        lax.fori_loop(0, SEQ // NPB, iter_body, 0)
        for oq in range(NOB):
            wait_wb(oq)

    return emb


_emb = _build()


def kernel(token_ids, token_table, pos_table):
    ids_t = token_ids.astype(jnp.int32).T
    table_p = token_table.reshape(VOCAB_PAIRS, 2 * DIM)
    out = _emb(ids_t, table_p, pos_table)
    return out.transpose(2, 0, 1)


# speedup vs baseline: 1.1119x; 1.1119x over previous
import functools

import jax
import jax.numpy as jnp
from jax import lax
from jax.experimental import pallas as pl
from jax.experimental.pallas import tpu as pltpu
from jax.experimental.pallas import tpu_sc as plsc

BATCH = 4096
SEQ = 200
DIM = 64
CHUNK = 100
NROWS = BATCH * SEQ // CHUNK
NW = 32
CPW = NROWS // NW
NBUF = 8
AHEAD = 4
UNROLL = NBUF


def _build():
    mesh = plsc.VectorSubcoreMesh(core_axis_name="c", subcore_axis_name="s")

    @functools.partial(
        pl.kernel,
        mesh=mesh,
        out_type=jax.ShapeDtypeStruct((BATCH, SEQ, DIM), jnp.float32),
        scratch_types=[
            pltpu.VMEM((CPW, CHUNK), jnp.int32),
            pltpu.VMEM((SEQ, DIM), jnp.float32),
            [pltpu.VMEM((CHUNK, DIM), jnp.float32) for _ in range(NBUF)],
            [pltpu.SemaphoreType.DMA for _ in range(NBUF)],
            [pltpu.SemaphoreType.DMA for _ in range(NBUF)],
        ],
        compiler_params=pltpu.CompilerParams(use_tc_tiling_on_sc=False),
    )
    def emb(ids_hbm, table_hbm, pos_hbm, out_hbm, idx_v, pos_v, bufs, gsem, wsem):
        wid = lax.axis_index("s") * 2 + lax.axis_index("c")
        base = wid * CPW
        pltpu.sync_copy(pos_hbm, pos_v)
        pltpu.sync_copy(ids_hbm.at[pl.ds(base, CPW)], idx_v)

        def fire_gather(c, slot):
            pltpu.async_copy(table_hbm.at[idx_v.at[c]], bufs[slot], gsem[slot])

        def out_dst(c, p0):
            return out_hbm.at[(base + c) // 2, pl.ds(p0, CHUNK)]

        for q in range(AHEAD):
            fire_gather(q, q)

        def iter_body(i, carry):
            c0 = i * UNROLL
            for q in range(UNROLL):
                c = c0 + q
                cn = c + AHEAD
                slot_n = (q + AHEAD) % NBUF
                p0n = ((q + AHEAD) & 1) * CHUNK

                def fire_next(cn=cn, slot_n=slot_n):
                    fire_gather(cn, slot_n)

                def wait_then_fire(cn=cn, slot_n=slot_n, p0n=p0n):
                    pltpu.make_async_copy(
                        bufs[slot_n], out_dst(0, p0n), wsem[slot_n]
                    ).wait()
                    fire_gather(cn, slot_n)

                if q + AHEAD < NBUF:
                    @pl.when(cn < CPW)
                    def _():
                        @pl.when(cn >= NBUF)
                        def _():
                            wait_then_fire()

                        @pl.when(cn < NBUF)
                        def _():
                            fire_next()

                else:
                    @pl.when(cn < CPW)
                    def _():
                        wait_then_fire()

                pltpu.make_async_copy(
                    table_hbm.at[idx_v.at[0]], bufs[q], gsem[q]
                ).wait()
                buf = bufs[q]
                p0 = (q & 1) * CHUNK

                @plsc.parallel_loop(0, CHUNK, step=2, unroll=4)
                def row_body(r):
                    for rr in range(2):
                        for d in range(DIM // 16):
                            sl = pl.ds(d * 16, 16)
                            buf[r + rr, sl] = buf[r + rr, sl] + pos_v[p0 + r + rr, sl]

                pltpu.async_copy(buf, out_dst(c, p0), wsem[q])
            return carry

        lax.fori_loop(0, CPW // UNROLL, iter_body, 0)
        for q in range(NBUF):
            pltpu.make_async_copy(
                bufs[q], out_dst(0, (q & 1) * CHUNK), wsem[q]).wait()

    return emb


_emb = _build()


def kernel(token_ids, token_table, pos_table):
    ids = token_ids.reshape(NROWS, CHUNK).astype(jnp.int32)
    return _emb(ids, token_table, pos_table)
